# baseline (speedup 1.0000x reference)
import functools

import jax
import jax.numpy as jnp
from jax import lax
from jax.experimental import pallas as pl
from jax.experimental.pallas import tpu as pltpu
from jax.experimental.pallas import tpu_sc as plsc


@functools.cache
def _make_broadcast_kernel(B, S, D, dtype):
    info = plsc.get_sparse_core_info()
    NC, NS = info.num_cores, info.num_subcores
    NW = NC * NS
    assert B % NW == 0
    b_per_w = B // NW
    mesh = plsc.VectorSubcoreMesh(core_axis_name="c", subcore_axis_name="s")

    @functools.partial(
        pl.kernel,
        mesh=mesh,
        out_type=jax.ShapeDtypeStruct((B, S, D), dtype),
        scratch_types=[
            pltpu.VMEM((S, D), dtype),
            pltpu.SemaphoreType.DMA,
            pltpu.SemaphoreType.DMA,
        ],
    )
    def k(table_hbm, out_hbm, tab_v, sem, load_sem):
        wid = lax.axis_index("s") * NC + lax.axis_index("c")
        base = wid * b_per_w
        nchunk = 4
        rows = S // nchunk
        loads = [
            pltpu.make_async_copy(
                table_hbm.at[pl.ds(j * rows, rows)],
                tab_v.at[pl.ds(j * rows, rows)],
                load_sem,
            )
            for j in range(nchunk)
        ]
        first = [
            pltpu.make_async_copy(
                tab_v.at[pl.ds(j * rows, rows)],
                out_hbm.at[base, pl.ds(j * rows, rows)],
                sem,
            )
            for j in range(nchunk)
        ]
        rest = [
            pltpu.make_async_copy(tab_v, out_hbm.at[base + i], sem)
            for i in range(1, b_per_w)
        ]
        for ld in loads:
            ld.start()
        for j in range(nchunk):
            loads[j].wait()
            first[j].start()
        for c in rest:
            c.start()
        for c in first:
            c.wait()
        for c in rest:
            c.wait()

    return k


def kernel(inputs, table):
    B, S = inputs.shape
    S2, D = table.shape
    assert S == S2
    return _make_broadcast_kernel(B, S, D, table.dtype)(table)


# speedup vs baseline: 1.0339x; 1.0339x over previous
import functools

import jax
import jax.numpy as jnp
from jax import lax
from jax.experimental import pallas as pl
from jax.experimental.pallas import tpu as pltpu
from jax.experimental.pallas import tpu_sc as plsc


@functools.cache
def _make_broadcast_kernel(B, S, D, dtype):
    info = plsc.get_sparse_core_info()
    NC, NS = info.num_cores, info.num_subcores
    NW = NC * NS
    assert B % NW == 0
    b_per_w = B // NW
    mesh = plsc.VectorSubcoreMesh(core_axis_name="c", subcore_axis_name="s")

    @functools.partial(
        pl.kernel,
        mesh=mesh,
        out_type=jax.ShapeDtypeStruct((B, S, D), dtype),
        scratch_types=[
            pltpu.VMEM((S, D), dtype),
            pltpu.SemaphoreType.DMA,
        ],
    )
    def k(table_hbm, out_hbm, tab_v, sem):
        wid = lax.axis_index("s") * NC + lax.axis_index("c")
        base = wid * b_per_w
        pltpu.sync_copy(table_hbm, tab_v)
        copies = [
            pltpu.make_async_copy(tab_v, out_hbm.at[base + i], sem)
            for i in range(b_per_w)
        ]
        for c in copies:
            c.start()
        for c in copies:
            c.wait()

    return k


def kernel(inputs, table):
    B, S = inputs.shape
    S2, D = table.shape
    assert S == S2
    return _make_broadcast_kernel(B, S, D, table.dtype)(table)
